# baseline (speedup 1.0000x reference)
import functools

import jax
import jax.numpy as jnp
from jax import lax
from jax.experimental import pallas as pl
from jax.experimental.pallas import tpu as pltpu
from jax.experimental.pallas import tpu_sc as plsc

N = 1024
M = 100000
B = 16384

NC = 2
NS = 16
LANES = 16
NW = NC * NS
B_PER_W = B // NW
CHUNK = 32
N_CHUNKS = B_PER_W // CHUNK

TM = 512
T_GRID = (M + TM - 1) // TM
M_PAD = T_GRID * TM


def _trans_lse_kernel(e_ref, t_ref, lse_ref, m_scr, s_scr):
    j = pl.program_id(0)
    blk = e_ref[...]
    t_ref[...] = blk.T

    cols = j * TM + lax.broadcasted_iota(jnp.int32, blk.shape, 1)
    blkm = jnp.where(cols < M, blk, -jnp.inf)
    bm = jnp.max(blkm, axis=1, keepdims=True)
    bs = jnp.sum(jnp.exp(blkm - bm), axis=1, keepdims=True)

    @pl.when(j == 0)
    def _():
        m_scr[...] = bm
        s_scr[...] = bs

    @pl.when(j > 0)
    def _():
        m_old = m_scr[...]
        s_old = s_scr[...]
        m_new = jnp.maximum(m_old, bm)
        s_scr[...] = s_old * jnp.exp(m_old - m_new) + bs * jnp.exp(bm - m_new)
        m_scr[...] = m_new

    @pl.when(j == T_GRID - 1)
    def _():
        lse_ref[...] = m_scr[...] + jnp.log(s_scr[...])


def _trans_lse_call(e):
    return pl.pallas_call(
        _trans_lse_kernel,
        grid=(T_GRID,),
        in_specs=[pl.BlockSpec((N, TM), lambda j: (0, j))],
        out_specs=[
            pl.BlockSpec((TM, N), lambda j: (j, 0)),
            pl.BlockSpec((N, 1), lambda j: (0, 0)),
        ],
        out_shape=[
            jax.ShapeDtypeStruct((M, N), jnp.float32),
            jax.ShapeDtypeStruct((N, 1), jnp.float32),
        ],
        scratch_shapes=[
            pltpu.VMEM((N, 1), jnp.float32),
            pltpu.VMEM((N, 1), jnp.float32),
        ],
    )(e)


def _sc_lookup(t_hbm, idx_hbm, lse_hbm, out_hbm, idx_v, lse_v, buf0, buf1,
               gsem0, gsem1, osem0, osem1):
    wid = lax.axis_index("s") * NC + lax.axis_index("c")
    pltpu.sync_copy(lse_hbm, lse_v)
    pltpu.sync_copy(idx_hbm.at[pl.ds(wid * N_CHUNKS, N_CHUNKS), :], idx_v)

    bufs = (buf0, buf1)
    gsems = (gsem0, gsem1)
    osems = (osem0, osem1)
    b0 = wid * B_PER_W
    lanes = lax.iota(jnp.int32, LANES)

    def start_gather(k, p):
        pltpu.async_copy(t_hbm.at[idx_v.at[k]], bufs[p], gsems[p])

    def wait_gather(k, p):
        pltpu.make_async_copy(t_hbm.at[idx_v.at[k]], bufs[p], gsems[p]).wait()

    def start_out(k, p):
        pltpu.async_copy(bufs[p], out_hbm.at[pl.ds(b0 + k * CHUNK, CHUNK), :],
                         osems[p])

    def wait_out(k, p):
        pltpu.make_async_copy(bufs[p],
                              out_hbm.at[pl.ds(b0 + k * CHUNK, CHUNK), :],
                              osems[p]).wait()

    start_gather(0, 0)

    @pl.loop(0, N_CHUNKS // 2)
    def _chunk2(k2):
        for par in range(2):
            kk = k2 * 2 + par
            q = 1 - par

            @pl.when(kk <= N_CHUNKS - 2)
            def _():
                @pl.when(kk >= 1)
                def _():
                    wait_out(kk - 1, q)
                start_gather(kk + 1, q)

            wait_gather(kk, par)
            buf = bufs[par]

            @pl.loop(0, N // LANES)
            def _v(v, buf=buf):
                lse16 = lse_v[pl.ds(v * LANES, LANES)]
                c16 = v * LANES + lanes

                @pl.loop(0, CHUNK, unroll=8)
                def _rows(r, buf=buf, lse16=lse16, c16=c16):
                    r16 = jnp.zeros((LANES,), jnp.int32) + r
                    vals = plsc.load_gather(buf, [r16, c16])
                    plsc.store_scatter(buf, [r16, c16], vals - lse16)

            start_out(kk, par)

    wait_out(N_CHUNKS - 2, 0)
    wait_out(N_CHUNKS - 1, 1)


def _sc_lookup_call(t, idx2d, lse):
    mesh = plsc.VectorSubcoreMesh(core_axis_name="c", subcore_axis_name="s")
    return pl.kernel(
        _sc_lookup,
        out_type=jax.ShapeDtypeStruct((B, N), jnp.float32),
        mesh=mesh,
        compiler_params=pltpu.CompilerParams(needs_layout_passes=False),
        scratch_types=[
            pltpu.VMEM((N_CHUNKS, CHUNK), jnp.int32),
            pltpu.VMEM((N,), jnp.float32),
            pltpu.VMEM((CHUNK, N), jnp.float32),
            pltpu.VMEM((CHUNK, N), jnp.float32),
            pltpu.SemaphoreType.DMA,
            pltpu.SemaphoreType.DMA,
            pltpu.SemaphoreType.DMA,
            pltpu.SemaphoreType.DMA,
        ],
    )(t, idx2d, lse)


@jax.jit
def kernel(x_t, unnormalized_emission_matrix):
    idx2d = x_t.astype(jnp.int32).reshape(NW * N_CHUNKS, CHUNK)
    t, lse = _trans_lse_call(unnormalized_emission_matrix)
    return _sc_lookup_call(t, idx2d, lse.reshape(N))
